# baseline (speedup 1.0000x reference)
import functools

import jax
import jax.numpy as jnp
from jax.experimental import pallas as pl
from jax.experimental.pallas import tpu as pltpu

N = 4096
D = 256
BM = 256
N_BM = N // BM
BSH = 512
N_SH = N // BSH
N_PAIRS = N_SH * N_SH


def _pairs_done_by(j):
    ready = jnp.where(j >= 1, ((j - 1) // 2 + 1) ** 2, 0)
    return jnp.maximum(0, jnp.minimum(jnp.minimum(ready, 4 * j), N_PAIRS))


def _body(l_ref, xb_ref, w02_ref, w1_ref, w2x2_ref, out_ref,
          lbc_ref, yb_ref, zacc_ref):
    j = pl.program_id(0)

    @pl.when(j == 0)
    def _init():
        zacc_ref[...] = jnp.zeros_like(zacc_ref)

    @pl.when(j < N_BM)
    def _stream():
        rows = pl.ds(j * BM, BM)
        l_blk = l_ref[...].astype(jnp.bfloat16)
        for kb in range(N_SH):
            lbc_ref[kb, rows, :] = l_blk[:, kb * BSH:(kb + 1) * BSH]
        y = jnp.dot(l_blk, xb_ref[...], preferred_element_type=jnp.float32)
        yb_ref[rows, :] = y.astype(jnp.bfloat16)
        out_ref[rows, :] = (
            jnp.dot(xb_ref[rows, :], w02_ref[...],
                    preferred_element_type=jnp.float32)
            + jnp.dot(y, w1_ref[...], preferred_element_type=jnp.float32)
        )

    def _pair(t, carry):
        s = jnp.zeros((), jnp.int32)
        for v in range(1, N_SH):
            s = s + (t >= v * v).astype(jnp.int32)
        r = t - s * s
        m = jnp.where(r <= s, s, r - s - 1)
        k = jnp.where(r <= s, r, s)
        a = lbc_ref[k, pl.ds(m * BSH, BSH), :]
        b = yb_ref[pl.ds(k * BSH, BSH), :]
        zrows = pl.ds(m * BSH, BSH)
        zacc_ref[zrows, :] += jnp.dot(a, b, preferred_element_type=jnp.float32)
        return carry

    lo = _pairs_done_by(j - 1)
    hi = _pairs_done_by(j)
    jax.lax.fori_loop(lo, hi, _pair, 0)

    @pl.when(j == N_BM)
    def _epilogue():
        zb = zacc_ref[...].astype(jnp.bfloat16)
        out_ref[...] += jnp.dot(zb, w2x2_ref[...],
                                preferred_element_type=jnp.float32)


@functools.partial(jax.jit, static_argnames=())
def _graph_conv(x, k, L, weight):
    scale = jnp.asarray(k, jnp.float32) / jnp.float32(weight.shape[0])
    w0 = weight[0] * scale
    w1 = weight[1] * scale
    w2 = weight[2] * scale
    w02 = w0 - w2
    w2x2 = 2.0 * w2
    xb = x.astype(jnp.bfloat16)

    grid = (N_BM + 1,)
    l_spec = pl.BlockSpec((BM, N), lambda j: (jnp.minimum(j, N_BM - 1), 0))
    full_spec = pl.BlockSpec((N, D), lambda j: (0, 0))
    w_spec = pl.BlockSpec((D, D), lambda j: (0, 0))

    out = pl.pallas_call(
        _body,
        grid=grid,
        in_specs=[l_spec, full_spec, w_spec, w_spec, w_spec],
        out_specs=pl.BlockSpec((N, D), lambda j: (0, 0)),
        out_shape=jax.ShapeDtypeStruct((N, D), jnp.float32),
        scratch_shapes=[
            pltpu.VMEM((N_SH, N, BSH), jnp.bfloat16),
            pltpu.VMEM((N, D), jnp.bfloat16),
            pltpu.VMEM((N, D), jnp.float32),
        ],
        compiler_params=pltpu.CompilerParams(
            dimension_semantics=("arbitrary",)),
    )(L, xb, w02, w1, w2x2)
    return out


def kernel(x, k, L, weight):
    return _graph_conv(x, k, L, weight)


# speedup vs baseline: 1.0085x; 1.0085x over previous
import functools

import jax
import jax.numpy as jnp
from jax.experimental import pallas as pl
from jax.experimental.pallas import tpu as pltpu

N = 4096
D = 256
BM = 256
N_BM = N // BM


def _body(la_ref, lo_ref, xb_ref, w02_ref, w1_ref, w2x2_ref, out_ref,
          lb_ref, yb_ref, part_ref):
    p = pl.program_id(0)
    i = pl.program_id(1)
    rows = pl.ds(i * BM, BM)

    def _phase0_work(l_blk_f32):
        l_blk = l_blk_f32.astype(jnp.bfloat16)
        lb_ref[rows, :] = l_blk
        y = jnp.dot(l_blk, xb_ref[...], preferred_element_type=jnp.float32)
        yb_ref[rows, :] = y.astype(jnp.bfloat16)
        part = (
            jnp.dot(xb_ref[rows, :], w02_ref[...],
                    preferred_element_type=jnp.float32)
            + jnp.dot(y, w1_ref[...], preferred_element_type=jnp.float32)
        )
        part_ref[rows, :] = part.astype(jnp.bfloat16)

    @pl.when((p == 0) & (i % 2 == 0))
    def _phase0_even():
        _phase0_work(la_ref[...])

    @pl.when((p == 0) & (i % 2 == 1))
    def _phase0_odd():
        _phase0_work(lo_ref[...])

    @pl.when(p == 1)
    def _phase1():
        z = jnp.dot(lb_ref[rows, :], yb_ref[...],
                    preferred_element_type=jnp.float32)
        out_ref[...] = part_ref[rows, :].astype(jnp.float32) + jnp.dot(
            z, w2x2_ref[...], preferred_element_type=jnp.float32)


@functools.partial(jax.jit, static_argnames=())
def _graph_conv(x, k, L, weight):
    scale = jnp.asarray(k, jnp.float32) / jnp.float32(weight.shape[0])
    w0 = weight[0] * scale
    w1 = weight[1] * scale
    w2 = weight[2] * scale
    w02 = w0 - w2
    w2x2 = 2.0 * w2
    xb = x.astype(jnp.bfloat16)

    grid = (2, N_BM)
    l_even_spec = pl.BlockSpec(
        (BM, N),
        lambda p, i: (jnp.where(p == 0, (i // 2) * 2, N_BM - 2), 0))
    l_odd_spec = pl.BlockSpec(
        (BM, N),
        lambda p, i: (jnp.where(p == 0, (i // 2) * 2 + 1, N_BM - 1), 0))
    full_spec = pl.BlockSpec((N, D), lambda p, i: (0, 0))
    out_spec = pl.BlockSpec(
        (BM, D), lambda p, i: (jnp.where(p == 0, 0, i), 0))
    w_spec = pl.BlockSpec((D, D), lambda p, i: (0, 0))

    out = pl.pallas_call(
        _body,
        grid=grid,
        in_specs=[l_even_spec, l_odd_spec, full_spec, w_spec, w_spec, w_spec],
        out_specs=out_spec,
        out_shape=jax.ShapeDtypeStruct((N, D), jnp.float32),
        scratch_shapes=[
            pltpu.VMEM((N, N), jnp.bfloat16),
            pltpu.VMEM((N, D), jnp.bfloat16),
            pltpu.VMEM((N, D), jnp.bfloat16),
        ],
        compiler_params=pltpu.CompilerParams(
            dimension_semantics=("arbitrary", "arbitrary")),
    )(L, L, xb, w02, w1, w2x2)
    return out


def kernel(x, k, L, weight):
    return _graph_conv(x, k, L, weight)


# speedup vs baseline: 1.3865x; 1.3748x over previous
import functools

import jax
import jax.numpy as jnp
from jax.experimental import pallas as pl
from jax.experimental.pallas import tpu as pltpu

N = 4096
D = 256
BM = 512
N_BM = N // BM
H = N // 2
BT = 1024


def _body(l_ref, xb_ref, w02_ref, w1_ref, w2x2_ref, out_ref,
          lba_ref, lbb_ref, yb_ref, part_ref, zacc_ref):
    p = pl.program_id(0)
    i = pl.program_id(1)
    rows = pl.ds(i * BM, BM)

    @pl.when(p == 0)
    def _phase0():
        l_blk = l_ref[...].astype(jnp.bfloat16)
        lba_ref[rows, :] = l_blk[:, :H]
        lbb_ref[rows, :] = l_blk[:, H:]
        y = jnp.dot(l_blk, xb_ref[...], preferred_element_type=jnp.float32)
        yb_ref[rows, :] = y.astype(jnp.bfloat16)
        part = (
            jnp.dot(xb_ref[rows, :], w02_ref[...],
                    preferred_element_type=jnp.float32)
            + jnp.dot(y, w1_ref[...], preferred_element_type=jnp.float32)
        )
        part_ref[rows, :] = part.astype(jnp.bfloat16)

    @pl.when((p == 0) & (i >= N_BM - N // BT))
    def _z_first_half():
        trows = pl.ds((i - (N_BM - N // BT)) * BT, BT)
        zh = jnp.dot(lba_ref[trows, :], yb_ref[:H, :],
                     preferred_element_type=jnp.float32)
        zacc_ref[trows, :] = zh.astype(jnp.bfloat16)

    @pl.when(p == 1)
    def _phase1():
        z = zacc_ref[rows, :].astype(jnp.float32) + jnp.dot(
            lbb_ref[rows, :], yb_ref[H:, :],
            preferred_element_type=jnp.float32)
        out_ref[...] = part_ref[rows, :].astype(jnp.float32) + jnp.dot(
            z, w2x2_ref[...], preferred_element_type=jnp.float32)


@functools.partial(jax.jit, static_argnames=())
def _graph_conv(x, k, L, weight):
    scale = jnp.asarray(k, jnp.float32) / jnp.float32(weight.shape[0])
    w0 = weight[0] * scale
    w1 = weight[1] * scale
    w2 = weight[2] * scale
    w02 = w0 - w2
    w2x2 = 2.0 * w2
    xb = x.astype(jnp.bfloat16)

    grid = (2, N_BM)
    l_spec = pl.BlockSpec(
        (BM, N), lambda p, i: (jnp.where(p == 0, i, N_BM - 1), 0))
    full_spec = pl.BlockSpec((N, D), lambda p, i: (0, 0))
    out_spec = pl.BlockSpec(
        (BM, D), lambda p, i: (jnp.where(p == 0, 0, i), 0))
    w_spec = pl.BlockSpec((D, D), lambda p, i: (0, 0))

    out = pl.pallas_call(
        _body,
        grid=grid,
        in_specs=[l_spec, full_spec, w_spec, w_spec, w_spec],
        out_specs=out_spec,
        out_shape=jax.ShapeDtypeStruct((N, D), jnp.float32),
        scratch_shapes=[
            pltpu.VMEM((N, H), jnp.bfloat16),
            pltpu.VMEM((N, H), jnp.bfloat16),
            pltpu.VMEM((N, D), jnp.bfloat16),
            pltpu.VMEM((N, D), jnp.bfloat16),
            pltpu.VMEM((N, D), jnp.bfloat16),
        ],
        compiler_params=pltpu.CompilerParams(
            dimension_semantics=("arbitrary", "arbitrary")),
    )(L, xb, w02, w1, w2x2)
    return out


def kernel(x, k, L, weight):
    return _graph_conv(x, k, L, weight)
